# Initial kernel scaffold; baseline (speedup 1.0000x reference)
import jax
import jax.numpy as jnp
from jax.experimental import pallas as pl


def kernel(x, noise_idx):
    raise NotImplementedError("write your pallas kernel here")


# speedup vs baseline: 44.2779x; 44.2779x over previous
import jax
import jax.numpy as jnp
from jax import lax
from jax.experimental import pallas as pl
from jax.experimental.pallas import tpu as pltpu
from jax.experimental.pallas import tpu_sc as plsc

B = 16384
N = 4096
K = 408
HALF = K // 2
SALT_VALUE = 1.0
PEPPER_VALUE = -1.0

L = 16
NC, NS = 2, 16
NW = NC * NS
ROWS_PER_W = B // NW
R = 8
NB = ROWS_PER_W // R
NBUF = 3
SALT_OFFS = tuple(range(0, HALF - 15, 15)) + (HALF - 15,)
PEP_OFFS = tuple(range(0, HALF - L, L)) + (HALF - L,)


def _sp_body(x_hbm, idx_hbm, o_hbm, xb0, xb1, xb2, ib0, ib1, ib2,
             six0, six1, six2, sii0, sii1, sii2, so0, so1, so2):
    wid = lax.axis_index("s") * NC + lax.axis_index("c")
    base = wid * ROWS_PER_W
    xbufs = (xb0, xb1, xb2)
    ibufs = (ib0, ib1, ib2)
    sin_x = (six0, six1, six2)
    sin_i = (sii0, sii1, sii2)
    souts = (so0, so1, so2)

    def start_in(j, b):
        jc = jnp.minimum(j, NB - 1)
        r0 = base + jc * R
        pltpu.make_async_copy(
            x_hbm.at[pl.ds(r0 * N, R * N)], xbufs[b], sin_x[b]).start()
        pltpu.make_async_copy(
            idx_hbm.at[pl.ds(r0 * K, R * K)], ibufs[b], sin_i[b]).start()

    def wait_in(b):
        pltpu.make_async_copy(
            x_hbm.at[pl.ds(0, R * N)], xbufs[b], sin_x[b]).wait()
        pltpu.make_async_copy(
            idx_hbm.at[pl.ds(0, R * K)], ibufs[b], sin_i[b]).wait()

    def start_out(j, b):
        r0 = base + j * R
        pltpu.make_async_copy(
            xbufs[b], o_hbm.at[pl.ds(r0 * N, R * N)], souts[b]).start()

    def wait_out(b):
        pltpu.make_async_copy(
            xbufs[b], o_hbm.at[pl.ds(0, R * N)], souts[b]).wait()

    salt = jnp.full((L,), SALT_VALUE, jnp.float32)
    pepper = jnp.full((L,), PEPPER_VALUE, jnp.float32)
    lane15_off = lax.iota(jnp.int32, L) < 15

    def scatter_batch(b):
        xb, ib = xbufs[b], ibufs[b]
        for ri in range(R):
            rbase = jnp.full((L,), ri * N, jnp.int32)
            for off in SALT_OFFS:
                iv = ib[pl.ds(ri * K + off, L)] + rbase
                plsc.store_scatter(xb, [iv], salt, mask=lane15_off)
        for ri in range(R):
            rbase = jnp.full((L,), ri * N, jnp.int32)
            for off in PEP_OFFS:
                iv = ib[pl.ds(ri * K + HALF + off, L)] + rbase
                plsc.store_scatter(xb, [iv], pepper)

    start_in(jnp.int32(0), 0)
    start_in(jnp.int32(1), 1)
    wait_in(0)
    scatter_batch(0)
    start_out(jnp.int32(0), 0)
    start_in(jnp.int32(2), 2)

    def body(g, carry):
        for b in range(NBUF):
            j = 1 + g * NBUF + b
            bb = (1 + b) % NBUF
            wait_in(bb)
            scatter_batch(bb)
            start_out(j, bb)
            wait_out(b)
            start_in(j + 2, b)
        return carry

    lax.fori_loop(0, (NB - 1) // NBUF, body, jnp.int32(0))
    wait_out((NB - 1) % NBUF)
    wait_in(NB % NBUF)
    wait_in((NB + 1) % NBUF)


def _build():
    mesh = plsc.VectorSubcoreMesh(
        core_axis_name="c", subcore_axis_name="s",
        num_cores=NC, num_subcores=NS)
    return pl.kernel(
        _sp_body,
        out_type=jax.ShapeDtypeStruct((B * N,), jnp.float32),
        mesh=mesh,
        compiler_params=pltpu.CompilerParams(needs_layout_passes=False),
        scratch_types=[
            pltpu.VMEM((R * N,), jnp.float32),
            pltpu.VMEM((R * N,), jnp.float32),
            pltpu.VMEM((R * N,), jnp.float32),
            pltpu.VMEM((R * K,), jnp.int32),
            pltpu.VMEM((R * K,), jnp.int32),
            pltpu.VMEM((R * K,), jnp.int32),
            pltpu.SemaphoreType.DMA,
            pltpu.SemaphoreType.DMA,
            pltpu.SemaphoreType.DMA,
            pltpu.SemaphoreType.DMA,
            pltpu.SemaphoreType.DMA,
            pltpu.SemaphoreType.DMA,
            pltpu.SemaphoreType.DMA,
            pltpu.SemaphoreType.DMA,
            pltpu.SemaphoreType.DMA,
        ],
    )


@jax.jit
def kernel(x, noise_idx):
    x_flat = x.reshape(B * N)
    idx_flat = noise_idx.reshape(B * K)
    return _build()(x_flat, idx_flat).reshape(B, N)


# speedup vs baseline: 44.3177x; 1.0009x over previous
import jax
import jax.numpy as jnp
from jax import lax
from jax.experimental import pallas as pl
from jax.experimental.pallas import tpu as pltpu
from jax.experimental.pallas import tpu_sc as plsc

B = 16384
N = 4096
K = 408
HALF = K // 2
SALT_VALUE = 1.0
PEPPER_VALUE = -1.0

L = 16
NC, NS = 2, 16
NW = NC * NS
ROWS_PER_W = B // NW
R = 8
NB = ROWS_PER_W // R
NBUF = 3
SALT_OFFS = tuple(range(0, HALF - 15, 15)) + (HALF - 15,)
PEP_OFFS = tuple(range(0, HALF - L, L)) + (HALF - L,)


def _sp_body(x_hbm, idx_hbm, o_hbm, xb0, xb1, xb2, ib0, ib1, ib2,
             six0, six1, six2, sii0, sii1, sii2, so0, so1, so2):
    wid = lax.axis_index("s") * NC + lax.axis_index("c")
    base = wid * ROWS_PER_W
    xbufs = (xb0, xb1, xb2)
    ibufs = (ib0, ib1, ib2)
    sin_x = (six0, six1, six2)
    sin_i = (sii0, sii1, sii2)
    souts = (so0, so1, so2)

    HN = R * N // 2

    def start_in(j, b):
        jc = jnp.minimum(j, NB - 1)
        r0 = base + jc * R
        pltpu.make_async_copy(
            x_hbm.at[pl.ds(r0 * N, HN)],
            xbufs[b].at[pl.ds(0, HN)], sin_x[b]).start()
        pltpu.make_async_copy(
            x_hbm.at[pl.ds(r0 * N + HN, HN)],
            xbufs[b].at[pl.ds(HN, HN)], sin_x[b]).start()
        pltpu.make_async_copy(
            idx_hbm.at[pl.ds(r0 * K, R * K)], ibufs[b], sin_i[b]).start()

    def wait_in(b):
        pltpu.make_async_copy(
            x_hbm.at[pl.ds(0, R * N)], xbufs[b], sin_x[b]).wait()
        pltpu.make_async_copy(
            idx_hbm.at[pl.ds(0, R * K)], ibufs[b], sin_i[b]).wait()

    def start_out(j, b):
        r0 = base + j * R
        pltpu.make_async_copy(
            xbufs[b].at[pl.ds(0, HN)],
            o_hbm.at[pl.ds(r0 * N, HN)], souts[b]).start()
        pltpu.make_async_copy(
            xbufs[b].at[pl.ds(HN, HN)],
            o_hbm.at[pl.ds(r0 * N + HN, HN)], souts[b]).start()

    def wait_out(b):
        pltpu.make_async_copy(
            xbufs[b], o_hbm.at[pl.ds(0, R * N)], souts[b]).wait()

    salt = jnp.full((L,), SALT_VALUE, jnp.float32)
    pepper = jnp.full((L,), PEPPER_VALUE, jnp.float32)
    lane15_off = lax.iota(jnp.int32, L) < 15

    def scatter_batch(b):
        xb, ib = xbufs[b], ibufs[b]
        for ri in range(R):
            rbase = jnp.full((L,), ri * N, jnp.int32)
            for off in SALT_OFFS:
                iv = ib[pl.ds(ri * K + off, L)] + rbase
                plsc.store_scatter(xb, [iv], salt, mask=lane15_off)
        for ri in range(R):
            rbase = jnp.full((L,), ri * N, jnp.int32)
            for off in PEP_OFFS:
                iv = ib[pl.ds(ri * K + HALF + off, L)] + rbase
                plsc.store_scatter(xb, [iv], pepper)

    start_in(jnp.int32(0), 0)
    start_in(jnp.int32(1), 1)
    wait_in(0)
    scatter_batch(0)
    start_out(jnp.int32(0), 0)
    start_in(jnp.int32(2), 2)

    def body(g, carry):
        for b in range(NBUF):
            j = 1 + g * NBUF + b
            bb = (1 + b) % NBUF
            wait_in(bb)
            scatter_batch(bb)
            start_out(j, bb)
            wait_out(b)
            start_in(j + 2, b)
        return carry

    lax.fori_loop(0, (NB - 1) // NBUF, body, jnp.int32(0))
    wait_out((NB - 1) % NBUF)
    wait_in(NB % NBUF)
    wait_in((NB + 1) % NBUF)


def _build():
    mesh = plsc.VectorSubcoreMesh(
        core_axis_name="c", subcore_axis_name="s",
        num_cores=NC, num_subcores=NS)
    return pl.kernel(
        _sp_body,
        out_type=jax.ShapeDtypeStruct((B * N,), jnp.float32),
        mesh=mesh,
        compiler_params=pltpu.CompilerParams(needs_layout_passes=False),
        scratch_types=[
            pltpu.VMEM((R * N,), jnp.float32),
            pltpu.VMEM((R * N,), jnp.float32),
            pltpu.VMEM((R * N,), jnp.float32),
            pltpu.VMEM((R * K,), jnp.int32),
            pltpu.VMEM((R * K,), jnp.int32),
            pltpu.VMEM((R * K,), jnp.int32),
            pltpu.SemaphoreType.DMA,
            pltpu.SemaphoreType.DMA,
            pltpu.SemaphoreType.DMA,
            pltpu.SemaphoreType.DMA,
            pltpu.SemaphoreType.DMA,
            pltpu.SemaphoreType.DMA,
            pltpu.SemaphoreType.DMA,
            pltpu.SemaphoreType.DMA,
            pltpu.SemaphoreType.DMA,
        ],
    )


@jax.jit
def kernel(x, noise_idx):
    x_flat = x.reshape(B * N)
    idx_flat = noise_idx.reshape(B * K)
    return _build()(x_flat, idx_flat).reshape(B, N)
